# baseline (speedup 1.0000x reference)
import functools

import jax
import jax.numpy as jnp
from jax import lax
from jax.experimental import pallas as pl
from jax.experimental.pallas import tpu as pltpu
from jax.experimental.pallas import tpu_sc as plsc

N = 10000
E = 320000
D = 128
CTX = 16
C = 40

NC = 2
NS = 16
NW = NC * NS
CH = 64
NCHUNK = 160
G = 16
NG = NCHUNK // G
EP = NW * NCHUNK * CH
NP = 10240
RPT = NP // NS
SENT = N
DW = 128

BLK = 256


def _sc_agg_body(h_hbm, src_hbm, dst_hbm, acc_out, srcv, dstv, bufa, bufb,
                 acc_sh, sem):
    c = lax.axis_index("c")
    s = lax.axis_index("s")
    wid = s * NC + c

    zeros16 = jnp.zeros((16,), jnp.float32)

    def zrow(i, carry):
        for v in range(D // 16):
            bufa[i, pl.ds(v * 16, 16)] = zeros16
        return carry
    lax.fori_loop(0, CH, zrow, 0)
    for b in range(RPT // CH):
        pltpu.sync_copy(bufa, acc_sh.at[pl.ds(s * RPT + b * CH, CH)])
    plsc.subcore_barrier()

    bufs = (bufa, bufb)

    def group(gi, carry):
        pltpu.sync_copy(src_hbm.at[wid].at[pl.ds(gi * G, G)], srcv)
        pltpu.sync_copy(dst_hbm.at[wid].at[pl.ds(gi * G, G)], dstv)

        cp = pltpu.async_copy(h_hbm.at[srcv.at[0]], bufa, sem)
        for j in range(G):
            cp.wait()
            if j + 1 < G:
                cp = pltpu.async_copy(h_hbm.at[srcv.at[j + 1]],
                                      bufs[(j + 1) % 2], sem)
            pltpu.sync_copy(bufs[j % 2], acc_sh.at[dstv.at[j]], add=True)
        return carry
    lax.fori_loop(0, NG, group, 0)
    plsc.subcore_barrier()

    for b in range(RPT // CH):
        pltpu.sync_copy(acc_sh.at[pl.ds(s * RPT + b * CH, CH)], bufa)
        pltpu.sync_copy(bufa, acc_out.at[c].at[pl.ds(s * RPT + b * CH, CH)])


@functools.lru_cache(maxsize=None)
def _make_sc_agg():
    mesh = plsc.VectorSubcoreMesh(core_axis_name="c", subcore_axis_name="s")
    return pl.kernel(
        _sc_agg_body,
        out_type=jax.ShapeDtypeStruct((NC, NP, D), jnp.float32),
        mesh=mesh,
        scratch_types=[
            pltpu.VMEM((G, CH), jnp.int32),
            pltpu.VMEM((G, CH), jnp.int32),
            pltpu.VMEM((CH, D), jnp.float32),
            pltpu.VMEM((CH, D), jnp.float32),
            pltpu.VMEM_SHARED((NP, D), jnp.float32),
            pltpu.SemaphoreType.DMA,
        ],
    )


def _sc_agg(*args):
    return _make_sc_agg()(*args)


def _sc_deg_body(dst_hbm, deg_out, dstv, ones_buf, deg_sh):
    c = lax.axis_index("c")
    s = lax.axis_index("s")
    wid = s * NC + c

    zeros16 = jnp.zeros((16,), jnp.float32)
    ones16 = jnp.ones((16,), jnp.float32)

    def zrow(i, carry):
        for v in range(DW // 16):
            ones_buf[i, pl.ds(v * 16, 16)] = zeros16
        return carry
    lax.fori_loop(0, CH, zrow, 0)
    for b in range(RPT // CH):
        pltpu.sync_copy(ones_buf, deg_sh.at[pl.ds(s * RPT + b * CH, CH)])

    def orow(i, carry):
        for v in range(DW // 16):
            ones_buf[i, pl.ds(v * 16, 16)] = ones16
        return carry
    lax.fori_loop(0, CH, orow, 0)
    plsc.subcore_barrier()

    def group(gi, carry):
        pltpu.sync_copy(dst_hbm.at[wid].at[pl.ds(gi * G, G)], dstv)

        def chunk(j, carry2):
            pltpu.sync_copy(ones_buf, deg_sh.at[dstv.at[j]], add=True)
            return carry2
        lax.fori_loop(0, G, chunk, 0)
        return carry
    lax.fori_loop(0, NG, group, 0)
    plsc.subcore_barrier()

    for b in range(RPT // CH):
        pltpu.sync_copy(deg_sh.at[pl.ds(s * RPT + b * CH, CH)], ones_buf)
        pltpu.sync_copy(ones_buf, deg_out.at[c].at[pl.ds(s * RPT + b * CH, CH)])


@functools.lru_cache(maxsize=None)
def _make_sc_deg():
    mesh = plsc.VectorSubcoreMesh(core_axis_name="c", subcore_axis_name="s")
    return pl.kernel(
        _sc_deg_body,
        out_type=jax.ShapeDtypeStruct((NC, NP, DW), jnp.float32),
        mesh=mesh,
        scratch_types=[
            pltpu.VMEM((G, CH), jnp.int32),
            pltpu.VMEM((CH, DW), jnp.float32),
            pltpu.VMEM_SHARED((NP, DW), jnp.float32),
        ],
    )


def _sc_deg(*args):
    return _make_sc_deg()(*args)


def _tc_dense_body(p_ref, d_ref, h_ref, wl, bl, wr, g_ref, be_ref, o_ref):
    deg = jnp.maximum(d_ref[0, :, :1] + d_ref[1, :, :1], 1.0)
    agg = (p_ref[0] + p_ref[1]) / deg
    t = (jnp.dot(agg, wl[...], preferred_element_type=jnp.float32)
         + jnp.dot(h_ref[...], wr[...], preferred_element_type=jnp.float32)
         + bl[...])
    mu = jnp.mean(t, axis=1, keepdims=True)
    var = jnp.mean((t - mu) ** 2, axis=1, keepdims=True)
    y = (t - mu) * lax.rsqrt(var + 1e-5) * g_ref[...] + be_ref[...]
    o_ref[...] = jnp.maximum(y, 0.0)


def _tc_dense(p, degp, h, W_l, b_l, W_r, g, be):
    full = lambda shape: pl.BlockSpec(shape, lambda b: tuple(0 for _ in shape))
    return pl.pallas_call(
        _tc_dense_body,
        grid=(NP // BLK,),
        in_specs=[
            pl.BlockSpec((NC, BLK, D), lambda b: (0, b, 0)),
            pl.BlockSpec((NC, BLK, DW), lambda b: (0, b, 0)),
            pl.BlockSpec((BLK, D), lambda b: (b, 0)),
            full((D, D)), full((1, D)), full((D, D)), full((1, D)), full((1, D)),
        ],
        out_specs=pl.BlockSpec((BLK, D), lambda b: (b, 0)),
        out_shape=jax.ShapeDtypeStruct((NP, D), jnp.float32),
    )(p, degp, h, W_l, b_l.reshape(1, D), W_r,
      g.reshape(1, D), be.reshape(1, D))


def _tc_head_body(h_ref, ctx_ref, wc1, bc1, wc2, bc2, w1a, w1b, bh1, w2, bh2,
                  o_ref):
    f32 = jnp.float32
    t = jnp.maximum(
        jnp.dot(ctx_ref[...], wc1[...], preferred_element_type=f32) + bc1[...],
        0.0)
    ctxf = jnp.dot(t, wc2[...], preferred_element_type=f32) + bc2[...]
    z = (jnp.dot(h_ref[...], w1a[...], preferred_element_type=f32)
         + jnp.dot(ctxf, w1b[...], preferred_element_type=f32)
         + bh1[...])
    z = jnp.maximum(z, 0.0)
    o_ref[...] = jnp.dot(z, w2[...], preferred_element_type=f32) + bh2[...]


def _tc_head(h, ctx, Wc1, bc1, Wc2, bc2, Wh1a, Wh1b, bh1, Wh2p, bh2p):
    full = lambda shape: pl.BlockSpec(shape, lambda b: tuple(0 for _ in shape))
    return pl.pallas_call(
        _tc_head_body,
        grid=(NP // BLK,),
        in_specs=[
            pl.BlockSpec((BLK, D), lambda b: (b, 0)),
            pl.BlockSpec((BLK, CTX), lambda b: (b, 0)),
            full((CTX, CTX)), full((1, CTX)), full((CTX, D)), full((1, D)),
            full((D, D)), full((D, D)), full((1, D)),
            full((D, D)), full((1, D)),
        ],
        out_specs=pl.BlockSpec((BLK, D), lambda b: (b, 0)),
        out_shape=jax.ShapeDtypeStruct((NP, D), jnp.float32),
    )(h, ctx, Wc1, bc1.reshape(1, CTX), Wc2, bc2.reshape(1, D),
      Wh1a, Wh1b, bh1.reshape(1, D), Wh2p, bh2p)


def kernel(x, edge_index, ctx_nodes, W_l0, b_l0, W_r0, g0, be0,
           W_l1, b_l1, W_r1, g1, be1, Wc1, bc1, Wc2, bc2,
           Wh1, bh1, Wh2, bh2):
    f32 = jnp.float32
    x_pad = jnp.concatenate([x, jnp.zeros((NP - N, D), f32)])
    ctx_pad = jnp.concatenate([ctx_nodes, jnp.zeros((NP - N, CTX), f32)])
    npad = EP - E
    ppw = npad // NW
    pad_src = ((jnp.arange(npad, dtype=jnp.int32) * 37) % N).reshape(NW, ppw)
    pad_dst = (N + (jnp.arange(npad, dtype=jnp.int32) % (NP - N))).reshape(NW, ppw)
    src = jnp.concatenate([edge_index[0].reshape(NW, E // NW), pad_src],
                          axis=1).reshape(NW, NCHUNK, CH)
    dst = jnp.concatenate([edge_index[1].reshape(NW, E // NW), pad_dst],
                          axis=1).reshape(NW, NCHUNK, CH)

    Wh1a = Wh1[:D]
    Wh1b = Wh1[D:]
    Wh2p = jnp.concatenate([Wh2, jnp.zeros((D, D - C), f32)], axis=1)
    bh2p = jnp.concatenate([bh2, jnp.zeros((D - C,), f32)]).reshape(1, D)

    degp = _sc_deg(dst)
    p0 = _sc_agg(x_pad, src, dst)
    h1 = _tc_dense(p0, degp, x_pad, W_l0, b_l0, W_r0, g0, be0)
    p1 = _sc_agg(h1, src, dst)
    h2 = _tc_dense(p1, degp, h1, W_l1, b_l1, W_r1, g1, be1)
    logits = _tc_head(h2, ctx_pad, Wc1, bc1, Wc2, bc2, Wh1a, Wh1b, bh1,
                      Wh2p, bh2p)
    return logits[:N, :C]


# speedup vs baseline: 1.2814x; 1.2814x over previous
import functools

import jax
import jax.numpy as jnp
from jax import lax
from jax.experimental import pallas as pl
from jax.experimental.pallas import tpu as pltpu
from jax.experimental.pallas import tpu_sc as plsc

N = 10000
E = 320000
D = 128
CTX = 16
C = 40

NC = 2
NS = 16
NW = NC * NS
CH = 128
NCHUNK = 80
G = 8
NG = NCHUNK // G
EP = NW * NCHUNK * CH
NP = 10240
RPT = NP // NS
SENT = N
DW = 128

BLK = 256


def _sc_agg_body(h_hbm, src_hbm, dst_hbm, acc_out, srcv, dstv, bufa, bufb,
                 acc_sh, sem):
    c = lax.axis_index("c")
    s = lax.axis_index("s")
    wid = s * NC + c

    zeros16 = jnp.zeros((16,), jnp.float32)

    def zrow(i, carry):
        for v in range(D // 16):
            bufa[i, pl.ds(v * 16, 16)] = zeros16
        return carry
    lax.fori_loop(0, CH, zrow, 0)
    for b in range(RPT // CH):
        pltpu.sync_copy(bufa, acc_sh.at[pl.ds(s * RPT + b * CH, CH)])
    plsc.subcore_barrier()

    def group(gi, carry):
        pltpu.sync_copy(src_hbm.at[wid].at[pl.ds(gi * G, G)], srcv)
        pltpu.sync_copy(dst_hbm.at[wid].at[pl.ds(gi * G, G)], dstv)

        cpa0 = pltpu.async_copy(h_hbm.at[srcv.at[0]], bufa, sem)

        def pair(k, carry2):
            pltpu.make_async_copy(h_hbm.at[srcv.at[2 * k]], bufa, sem).wait()
            cpb = pltpu.async_copy(h_hbm.at[srcv.at[2 * k + 1]], bufb, sem)
            pltpu.sync_copy(bufa, acc_sh.at[dstv.at[2 * k]], add=True)

            @pl.when(k < G // 2 - 1)
            def _():
                pltpu.async_copy(h_hbm.at[srcv.at[2 * k + 2]], bufa, sem)
            cpb.wait()
            pltpu.sync_copy(bufb, acc_sh.at[dstv.at[2 * k + 1]], add=True)
            return carry2
        lax.fori_loop(0, G // 2, pair, 0)
        return carry
    lax.fori_loop(0, NG, group, 0)
    plsc.subcore_barrier()

    for b in range(RPT // CH):
        pltpu.sync_copy(acc_sh.at[pl.ds(s * RPT + b * CH, CH)], bufa)
        pltpu.sync_copy(bufa, acc_out.at[c].at[pl.ds(s * RPT + b * CH, CH)])


@functools.lru_cache(maxsize=None)
def _make_sc_agg():
    mesh = plsc.VectorSubcoreMesh(core_axis_name="c", subcore_axis_name="s")
    return pl.kernel(
        _sc_agg_body,
        out_type=jax.ShapeDtypeStruct((NC, NP, D), jnp.float32),
        mesh=mesh,
        scratch_types=[
            pltpu.VMEM((G, CH), jnp.int32),
            pltpu.VMEM((G, CH), jnp.int32),
            pltpu.VMEM((CH, D), jnp.float32),
            pltpu.VMEM((CH, D), jnp.float32),
            pltpu.VMEM_SHARED((NP, D), jnp.float32),
            pltpu.SemaphoreType.DMA,
        ],
    )


def _sc_agg(*args):
    return _make_sc_agg()(*args)


def _sc_deg_body(dst_hbm, deg_out, dstv, ones_buf, deg_sh):
    c = lax.axis_index("c")
    s = lax.axis_index("s")
    wid = s * NC + c

    zeros16 = jnp.zeros((16,), jnp.float32)
    ones16 = jnp.ones((16,), jnp.float32)

    def zrow(i, carry):
        for v in range(DW // 16):
            ones_buf[i, pl.ds(v * 16, 16)] = zeros16
        return carry
    lax.fori_loop(0, CH, zrow, 0)
    for b in range(RPT // CH):
        pltpu.sync_copy(ones_buf, deg_sh.at[pl.ds(s * RPT + b * CH, CH)])

    def orow(i, carry):
        for v in range(DW // 16):
            ones_buf[i, pl.ds(v * 16, 16)] = ones16
        return carry
    lax.fori_loop(0, CH, orow, 0)
    plsc.subcore_barrier()

    def group(gi, carry):
        pltpu.sync_copy(dst_hbm.at[wid].at[pl.ds(gi * G, G)], dstv)

        def chunk(j, carry2):
            pltpu.sync_copy(ones_buf, deg_sh.at[dstv.at[j]], add=True)
            return carry2
        lax.fori_loop(0, G, chunk, 0)
        return carry
    lax.fori_loop(0, NG, group, 0)
    plsc.subcore_barrier()

    for b in range(RPT // CH):
        pltpu.sync_copy(deg_sh.at[pl.ds(s * RPT + b * CH, CH)], ones_buf)
        pltpu.sync_copy(ones_buf, deg_out.at[c].at[pl.ds(s * RPT + b * CH, CH)])


@functools.lru_cache(maxsize=None)
def _make_sc_deg():
    mesh = plsc.VectorSubcoreMesh(core_axis_name="c", subcore_axis_name="s")
    return pl.kernel(
        _sc_deg_body,
        out_type=jax.ShapeDtypeStruct((NC, NP, DW), jnp.float32),
        mesh=mesh,
        scratch_types=[
            pltpu.VMEM((G, CH), jnp.int32),
            pltpu.VMEM((CH, DW), jnp.float32),
            pltpu.VMEM_SHARED((NP, DW), jnp.float32),
        ],
    )


def _sc_deg(*args):
    return _make_sc_deg()(*args)


def _tc_dense_body(p_ref, d_ref, h_ref, wl, bl, wr, g_ref, be_ref, o_ref):
    deg = jnp.maximum(d_ref[0, :, :1] + d_ref[1, :, :1], 1.0)
    agg = (p_ref[0] + p_ref[1]) / deg
    t = (jnp.dot(agg, wl[...], preferred_element_type=jnp.float32)
         + jnp.dot(h_ref[...], wr[...], preferred_element_type=jnp.float32)
         + bl[...])
    mu = jnp.mean(t, axis=1, keepdims=True)
    var = jnp.mean((t - mu) ** 2, axis=1, keepdims=True)
    y = (t - mu) * lax.rsqrt(var + 1e-5) * g_ref[...] + be_ref[...]
    o_ref[...] = jnp.maximum(y, 0.0)


def _tc_dense(p, degp, h, W_l, b_l, W_r, g, be):
    full = lambda shape: pl.BlockSpec(shape, lambda b: tuple(0 for _ in shape))
    return pl.pallas_call(
        _tc_dense_body,
        grid=(NP // BLK,),
        in_specs=[
            pl.BlockSpec((NC, BLK, D), lambda b: (0, b, 0)),
            pl.BlockSpec((NC, BLK, DW), lambda b: (0, b, 0)),
            pl.BlockSpec((BLK, D), lambda b: (b, 0)),
            full((D, D)), full((1, D)), full((D, D)), full((1, D)), full((1, D)),
        ],
        out_specs=pl.BlockSpec((BLK, D), lambda b: (b, 0)),
        out_shape=jax.ShapeDtypeStruct((NP, D), jnp.float32),
    )(p, degp, h, W_l, b_l.reshape(1, D), W_r,
      g.reshape(1, D), be.reshape(1, D))


def _tc_head_body(h_ref, ctx_ref, wc1, bc1, wc2, bc2, w1a, w1b, bh1, w2, bh2,
                  o_ref):
    f32 = jnp.float32
    t = jnp.maximum(
        jnp.dot(ctx_ref[...], wc1[...], preferred_element_type=f32) + bc1[...],
        0.0)
    ctxf = jnp.dot(t, wc2[...], preferred_element_type=f32) + bc2[...]
    z = (jnp.dot(h_ref[...], w1a[...], preferred_element_type=f32)
         + jnp.dot(ctxf, w1b[...], preferred_element_type=f32)
         + bh1[...])
    z = jnp.maximum(z, 0.0)
    o_ref[...] = jnp.dot(z, w2[...], preferred_element_type=f32) + bh2[...]


def _tc_head(h, ctx, Wc1, bc1, Wc2, bc2, Wh1a, Wh1b, bh1, Wh2p, bh2p):
    full = lambda shape: pl.BlockSpec(shape, lambda b: tuple(0 for _ in shape))
    return pl.pallas_call(
        _tc_head_body,
        grid=(NP // BLK,),
        in_specs=[
            pl.BlockSpec((BLK, D), lambda b: (b, 0)),
            pl.BlockSpec((BLK, CTX), lambda b: (b, 0)),
            full((CTX, CTX)), full((1, CTX)), full((CTX, D)), full((1, D)),
            full((D, D)), full((D, D)), full((1, D)),
            full((D, D)), full((1, D)),
        ],
        out_specs=pl.BlockSpec((BLK, D), lambda b: (b, 0)),
        out_shape=jax.ShapeDtypeStruct((NP, D), jnp.float32),
    )(h, ctx, Wc1, bc1.reshape(1, CTX), Wc2, bc2.reshape(1, D),
      Wh1a, Wh1b, bh1.reshape(1, D), Wh2p, bh2p)


def kernel(x, edge_index, ctx_nodes, W_l0, b_l0, W_r0, g0, be0,
           W_l1, b_l1, W_r1, g1, be1, Wc1, bc1, Wc2, bc2,
           Wh1, bh1, Wh2, bh2):
    f32 = jnp.float32
    x_pad = jnp.concatenate([x, jnp.zeros((NP - N, D), f32)])
    ctx_pad = jnp.concatenate([ctx_nodes, jnp.zeros((NP - N, CTX), f32)])
    npad = EP - E
    ppw = npad // NW
    pad_src = ((jnp.arange(npad, dtype=jnp.int32) * 37) % N).reshape(NW, ppw)
    pad_dst = (N + (jnp.arange(npad, dtype=jnp.int32) % (NP - N))).reshape(NW, ppw)
    src = jnp.concatenate([edge_index[0].reshape(NW, E // NW), pad_src],
                          axis=1).reshape(NW, NCHUNK, CH)
    dst = jnp.concatenate([edge_index[1].reshape(NW, E // NW), pad_dst],
                          axis=1).reshape(NW, NCHUNK, CH)

    Wh1a = Wh1[:D]
    Wh1b = Wh1[D:]
    Wh2p = jnp.concatenate([Wh2, jnp.zeros((D, D - C), f32)], axis=1)
    bh2p = jnp.concatenate([bh2, jnp.zeros((D - C,), f32)]).reshape(1, D)

    degp = _sc_deg(dst)
    p0 = _sc_agg(x_pad, src, dst)
    h1 = _tc_dense(p0, degp, x_pad, W_l0, b_l0, W_r0, g0, be0)
    p1 = _sc_agg(h1, src, dst)
    h2 = _tc_dense(p1, degp, h1, W_l1, b_l1, W_r1, g1, be1)
    logits = _tc_head(h2, ctx_pad, Wc1, bc1, Wc2, bc2, Wh1a, Wh1b, bh1,
                      Wh2p, bh2p)
    return logits[:N, :C]
